# baseline (speedup 1.0000x reference)
import jax
import jax.numpy as jnp
from jax import lax
from jax.experimental import pallas as pl
from jax.experimental.pallas import tpu as pltpu
from jax.experimental.pallas import tpu_sc as plsc

HIDDEN = 2048
LATENT = 16384
TOPK = 64
NTOK = 2048

BM = 256
LC = 1024
BR = 128
KC = 512

NC = 2
NS = 16
L = 16
NW = NC * NS
ROWS_PER_W = NTOK // NW
NV = LATENT // L
MGROUP = 16
NM = NV // MGROUP
SEG = 32


def _enc_body(x_ref, w_ref, out_ref):
    out_ref[...] = jax.lax.dot_general(
        x_ref[...], w_ref[...],
        (((1,), (1,)), ((), ())),
        preferred_element_type=jnp.float32,
        precision=jax.lax.Precision.DEFAULT)


def _tau_body(z_hbm, tau_hbm, zrow, mbuf, cand, taubuf):
    c = lax.axis_index("c")
    s = lax.axis_index("s")
    wid = s * NC + c
    base = wid * ROWS_PER_W
    lanes = lax.iota(jnp.int32, L)

    def row_loop(r, carry):
        pltpu.sync_copy(z_hbm.at[base + r], zrow)

        def mgroup(g, carry2):
            def inner(k, acc):
                return jnp.maximum(acc, zrow[pl.ds((g * MGROUP + k) * L, L)])
            m = lax.fori_loop(1, MGROUP, inner, zrow[pl.ds(g * MGROUP * L, L)])
            mbuf[pl.ds(g * L, L)] = m
            return 0
        lax.fori_loop(0, NM, mgroup, 0)

        def rmx(g, acc):
            return jnp.maximum(acc, mbuf[pl.ds(g * L, L)])
        rv = lax.fori_loop(1, NM, rmx, mbuf[pl.ds(0, L)])
        for d in (1, 2, 4, 8):
            rv = jnp.maximum(rv, rv[lanes ^ d])
        rmaxv = rv

        def lane_sum(v):
            for d in (1, 2, 4, 8):
                v = v + v[lanes ^ d]
            return v

        one = jnp.ones((L,), jnp.int32)
        zero = jnp.zeros((L,), jnp.int32)

        def mcount(tv):
            def cb(g, acc):
                return acc + jnp.where(mbuf[pl.ds(g * L, L)] >= tv, one, zero)
            return lane_sum(lax.fori_loop(0, NM, cb, zero))

        def mb_body(i, st):
            lo, hi = st
            mid = 0.5 * (lo + hi)
            pred = mcount(mid) >= TOPK
            return (jnp.where(pred, mid, lo), jnp.where(pred, hi, mid))

        tau0v, _ = lax.fori_loop(
            0, 12, mb_body, (jnp.zeros((L,), jnp.float32), rmaxv + 0.5))

        def czero(v, carry2):
            cand[pl.ds(v * L, L)] = jnp.zeros((L,), jnp.float32)
            return 0
        lax.fori_loop(0, SEG, czero, 0)

        segbase = lanes * SEG

        def compact(v, cnt):
            x = zrow[pl.ds(v * L, L)]
            mask = x >= tau0v
            ok = jnp.logical_and(mask, cnt < SEG)
            plsc.store_scatter(cand, [segbase + cnt], x, mask=ok)
            return cnt + jnp.where(mask, one, zero)

        lax.fori_loop(0, NV, compact, zero)

        def dcount(tv):
            def db(v, acc):
                return acc + jnp.where(cand[pl.ds(v * L, L)] >= tv, one, zero)
            return lane_sum(lax.fori_loop(0, SEG, db, zero))

        def d_body(i, st):
            lo, hi = st
            mid = 0.5 * (lo + hi)
            pred = dcount(mid) >= TOPK
            return (jnp.where(pred, mid, lo), jnp.where(pred, hi, mid))

        tauv, _ = lax.fori_loop(0, 30, d_body, (tau0v, rmaxv + 0.5))

        plsc.store_scatter(taubuf, [jnp.full((L,), r, jnp.int32)],
                           tauv, mask=lanes == 0)
        return 0

    lax.fori_loop(0, ROWS_PER_W, row_loop, 0)
    pltpu.sync_copy(taubuf, tau_hbm.at[pl.ds(base, ROWS_PER_W)])


def _mask_body(z_ref, tau_ref, out_ref):
    i = pl.program_id(0)
    tau = tau_ref[pl.ds(i * BR, BR), :]
    z = z_ref[...]
    out_ref[...] = jnp.where(z >= tau, jnp.maximum(z, 0.0), 0.0)


def _dec_body(a_ref, b_ref, out_ref):
    k = pl.program_id(0)
    r = pl.program_id(1)
    rows = pl.ds(r * BM, BM)

    @pl.when(k == 0)
    def _():
        out_ref[rows, :] = jnp.zeros((BM, HIDDEN), jnp.float32)

    out_ref[rows, :] += jax.lax.dot_general(
        a_ref[...], b_ref[...],
        (((1,), (1,)), ((), ())),
        preferred_element_type=jnp.float32,
        precision=jax.lax.Precision.DEFAULT)


def kernel(x, W_enc, W_dec):
    z = pl.pallas_call(
        _enc_body,
        grid=(LATENT // LC, NTOK // BM),
        in_specs=[pl.BlockSpec((BM, HIDDEN), lambda j, r: (r, 0)),
                  pl.BlockSpec((LC, HIDDEN), lambda j, r: (j, 0))],
        out_specs=pl.BlockSpec((BM, LC), lambda j, r: (r, j)),
        out_shape=jax.ShapeDtypeStruct((NTOK, LATENT), jnp.float32),
    )(x, W_enc)

    mesh = plsc.VectorSubcoreMesh(core_axis_name="c", subcore_axis_name="s")
    tau = pl.kernel(
        _tau_body,
        out_type=jax.ShapeDtypeStruct((NTOK,), jnp.float32),
        mesh=mesh,
        compiler_params=pltpu.CompilerParams(needs_layout_passes=False),
        scratch_types=[
            pltpu.VMEM((LATENT,), jnp.float32),
            pltpu.VMEM((NM * L,), jnp.float32),
            pltpu.VMEM((L * SEG,), jnp.float32),
            pltpu.VMEM((ROWS_PER_W,), jnp.float32),
        ],
    )(z)

    z_sparse = pl.pallas_call(
        _mask_body,
        grid=(NTOK // BR,),
        in_specs=[pl.BlockSpec((BR, LATENT), lambda i: (i, 0)),
                  pl.BlockSpec((NTOK, 1), lambda i: (0, 0))],
        out_specs=pl.BlockSpec((BR, LATENT), lambda i: (i, 0)),
        out_shape=jax.ShapeDtypeStruct((NTOK, LATENT), jnp.float32),
    )(z, tau.reshape(NTOK, 1))

    x_hat = pl.pallas_call(
        _dec_body,
        grid=(LATENT // KC, NTOK // BM),
        in_specs=[pl.BlockSpec((BM, KC), lambda k, r: (r, k)),
                  pl.BlockSpec((HIDDEN, KC), lambda k, r: (0, k))],
        out_specs=pl.BlockSpec((NTOK, HIDDEN), lambda k, r: (0, 0)),
        out_shape=jax.ShapeDtypeStruct((NTOK, HIDDEN), jnp.float32),
    )(z_sparse, W_dec)
    return x_hat, z_sparse


# speedup vs baseline: 1.1247x; 1.1247x over previous
import jax
import jax.numpy as jnp
from jax import lax
from jax.experimental import pallas as pl
from jax.experimental.pallas import tpu as pltpu
from jax.experimental.pallas import tpu_sc as plsc

HIDDEN = 2048
LATENT = 16384
TOPK = 64
NTOK = 2048

BM = 256
LC = 1024
BR = 128
KC = 512

NC = 2
NS = 16
L = 16
NW = NC * NS
ROWS_PER_W = NTOK // NW
NV = LATENT // L
MGROUP = 32
NM = NV // MGROUP
SEG = 32


def _enc_body(x_ref, w_ref, out_ref):
    out_ref[...] = jax.lax.dot_general(
        x_ref[...], w_ref[...],
        (((1,), (1,)), ((), ())),
        preferred_element_type=jnp.float32,
        precision=jax.lax.Precision.DEFAULT)


def _tau_body(z_hbm, tau_hbm, zrow, cand, taubuf):
    c = lax.axis_index("c")
    s = lax.axis_index("s")
    wid = s * NC + c
    base = wid * ROWS_PER_W
    lanes = lax.iota(jnp.int32, L)

    one = jnp.ones((L,), jnp.int32)
    zero = jnp.zeros((L,), jnp.int32)

    def lane_sum(v):
        for d in (1, 2, 4, 8):
            v = v + v[lanes ^ d]
        return v

    def row_loop(r, carry):
        pltpu.sync_copy(z_hbm.at[base + r], zrow)

        def build_m(g):
            def inner(k, acc):
                return jnp.maximum(acc, zrow[pl.ds((g * MGROUP + k) * L, L)])
            return lax.fori_loop(1, MGROUP, inner,
                                 zrow[pl.ds(g * MGROUP * L, L)], unroll=8)
        m_regs = tuple(build_m(g) for g in range(NM))

        rv = m_regs[0]
        for g in range(1, NM):
            rv = jnp.maximum(rv, m_regs[g])
        for d in (1, 2, 4, 8):
            rv = jnp.maximum(rv, rv[lanes ^ d])
        rmaxv = rv

        def mb_body(i, st):
            lo, hi = st
            mid = 0.5 * (lo + hi)
            acc = zero
            for g in range(NM):
                acc = acc + jnp.where(m_regs[g] >= mid, one, zero)
            pred = lane_sum(acc) >= TOPK
            return (jnp.where(pred, mid, lo), jnp.where(pred, hi, mid))

        tau0v, _ = lax.fori_loop(
            0, 12, mb_body, (jnp.zeros((L,), jnp.float32), rmaxv + 0.5))

        for v in range(SEG):
            cand[pl.ds(v * L, L)] = jnp.zeros((L,), jnp.float32)

        segbase = lanes * SEG

        def compact(v, cnt):
            x = zrow[pl.ds(v * L, L)]
            mask = x >= tau0v
            ok = jnp.logical_and(mask, cnt < SEG)
            plsc.store_scatter(cand, [segbase + cnt], x, mask=ok)
            return cnt + jnp.where(mask, one, zero)

        lax.fori_loop(0, NV, compact, zero, unroll=8)

        c_regs = tuple(cand[pl.ds(v * L, L)] for v in range(SEG))

        def d_body(i, st):
            lo, hi = st
            mid = 0.5 * (lo + hi)
            acc = zero
            for v in range(SEG):
                acc = acc + jnp.where(c_regs[v] >= mid, one, zero)
            pred = lane_sum(acc) >= TOPK
            return (jnp.where(pred, mid, lo), jnp.where(pred, hi, mid))

        tauv, _ = lax.fori_loop(0, 30, d_body, (tau0v, rmaxv + 0.5))

        plsc.store_scatter(taubuf, [jnp.full((L,), r, jnp.int32)],
                           tauv, mask=lanes == 0)
        return 0

    lax.fori_loop(0, ROWS_PER_W, row_loop, 0)
    pltpu.sync_copy(taubuf, tau_hbm.at[pl.ds(base, ROWS_PER_W)])


def _mask_body(z_ref, tau_ref, out_ref):
    i = pl.program_id(0)
    tau = tau_ref[pl.ds(i * BR, BR), :]
    z = z_ref[...]
    out_ref[...] = jnp.where(z >= tau, jnp.maximum(z, 0.0), 0.0)


def _dec_body(a_ref, b_ref, out_ref):
    k = pl.program_id(0)
    r = pl.program_id(1)
    rows = pl.ds(r * BM, BM)

    @pl.when(k == 0)
    def _():
        out_ref[rows, :] = jnp.zeros((BM, HIDDEN), jnp.float32)

    out_ref[rows, :] += jax.lax.dot_general(
        a_ref[...], b_ref[...],
        (((1,), (1,)), ((), ())),
        preferred_element_type=jnp.float32,
        precision=jax.lax.Precision.DEFAULT)


def kernel(x, W_enc, W_dec):
    z = pl.pallas_call(
        _enc_body,
        grid=(LATENT // LC, NTOK // BM),
        in_specs=[pl.BlockSpec((BM, HIDDEN), lambda j, r: (r, 0)),
                  pl.BlockSpec((LC, HIDDEN), lambda j, r: (j, 0))],
        out_specs=pl.BlockSpec((BM, LC), lambda j, r: (r, j)),
        out_shape=jax.ShapeDtypeStruct((NTOK, LATENT), jnp.float32),
    )(x, W_enc)

    mesh = plsc.VectorSubcoreMesh(core_axis_name="c", subcore_axis_name="s")
    tau = pl.kernel(
        _tau_body,
        out_type=jax.ShapeDtypeStruct((NTOK,), jnp.float32),
        mesh=mesh,
        compiler_params=pltpu.CompilerParams(needs_layout_passes=False),
        scratch_types=[
            pltpu.VMEM((LATENT,), jnp.float32),
            pltpu.VMEM((L * SEG,), jnp.float32),
            pltpu.VMEM((ROWS_PER_W,), jnp.float32),
        ],
    )(z)

    z_sparse = pl.pallas_call(
        _mask_body,
        grid=(NTOK // BR,),
        in_specs=[pl.BlockSpec((BR, LATENT), lambda i: (i, 0)),
                  pl.BlockSpec((NTOK, 1), lambda i: (0, 0))],
        out_specs=pl.BlockSpec((BR, LATENT), lambda i: (i, 0)),
        out_shape=jax.ShapeDtypeStruct((NTOK, LATENT), jnp.float32),
    )(z, tau.reshape(NTOK, 1))

    x_hat = pl.pallas_call(
        _dec_body,
        grid=(LATENT // KC, NTOK // BM),
        in_specs=[pl.BlockSpec((BM, KC), lambda k, r: (r, k)),
                  pl.BlockSpec((HIDDEN, KC), lambda k, r: (0, k))],
        out_specs=pl.BlockSpec((NTOK, HIDDEN), lambda k, r: (0, 0)),
        out_shape=jax.ShapeDtypeStruct((NTOK, HIDDEN), jnp.float32),
    )(z_sparse, W_dec)
    return x_hat, z_sparse


# speedup vs baseline: 1.8960x; 1.6858x over previous
import jax
import jax.numpy as jnp
from jax import lax
from jax.experimental import pallas as pl
from jax.experimental.pallas import tpu as pltpu
from jax.experimental.pallas import tpu_sc as plsc

HIDDEN = 2048
LATENT = 16384
TOPK = 64
NTOK = 2048

BM = 256
LC = 1024
BR = 128
KC = 512

NC = 2
NS = 16
L = 16
NW = NC * NS
ROWS_PER_W = NTOK // NW
NV = LATENT // L
MGROUP = 32
NM = NV // MGROUP
SEG = 32


def _enc_body(x_ref, w_ref, out_ref):
    out_ref[...] = jax.lax.dot_general(
        x_ref[...], w_ref[...],
        (((1,), (1,)), ((), ())),
        preferred_element_type=jnp.float32,
        precision=jax.lax.Precision.DEFAULT)


def _tau_body(z_hbm, tau_hbm, zrow, cand, taubuf):
    c = lax.axis_index("c")
    s = lax.axis_index("s")
    wid = s * NC + c
    base = wid * ROWS_PER_W
    lanes = lax.iota(jnp.int32, L)

    one = jnp.ones((L,), jnp.int32)
    zero = jnp.zeros((L,), jnp.int32)

    def lane_sum(v):
        for d in (1, 2, 4, 8):
            v = v + v[lanes ^ d]
        return v

    def row_loop(r, carry):
        pltpu.sync_copy(z_hbm.at[base + r], zrow)

        def build_m(g):
            def inner(k, acc):
                return jnp.maximum(acc, zrow[pl.ds(k * L, L)])
            return plsc.parallel_loop(
                g * MGROUP, (g + 1) * MGROUP, unroll=8,
                carry=jnp.full((L,), -1e30, jnp.float32))(inner)
        m_regs = tuple(build_m(g) for g in range(NM))

        rv = m_regs[0]
        for g in range(1, NM):
            rv = jnp.maximum(rv, m_regs[g])
        for d in (1, 2, 4, 8):
            rv = jnp.maximum(rv, rv[lanes ^ d])
        rmaxv = rv

        def mb_body(i, st):
            lo, hi = st
            mid = 0.5 * (lo + hi)
            acc = zero
            for g in range(NM):
                acc = acc + jnp.where(m_regs[g] >= mid, one, zero)
            pred = lane_sum(acc) >= TOPK
            return (jnp.where(pred, mid, lo), jnp.where(pred, hi, mid))

        tau0v, _ = lax.fori_loop(
            0, 12, mb_body, (jnp.zeros((L,), jnp.float32), rmaxv + 0.5))

        for v in range(SEG):
            cand[pl.ds(v * L, L)] = jnp.zeros((L,), jnp.float32)

        segbase = lanes * SEG

        def compact(v, cnt):
            x = zrow[pl.ds(v * L, L)]
            mask = x >= tau0v
            ok = jnp.logical_and(mask, cnt < SEG)
            plsc.store_scatter(cand, [segbase + cnt], x, mask=ok)
            return cnt + jnp.where(mask, one, zero)

        plsc.parallel_loop(0, NV, unroll=8, carry=zero)(compact)

        c_regs = tuple(cand[pl.ds(v * L, L)] for v in range(SEG))

        def d_body(i, st):
            lo, hi = st
            mid = 0.5 * (lo + hi)
            acc = zero
            for v in range(SEG):
                acc = acc + jnp.where(c_regs[v] >= mid, one, zero)
            pred = lane_sum(acc) >= TOPK
            return (jnp.where(pred, mid, lo), jnp.where(pred, hi, mid))

        tauv, _ = lax.fori_loop(0, 30, d_body, (tau0v, rmaxv + 0.5))

        plsc.store_scatter(taubuf, [jnp.full((L,), r, jnp.int32)],
                           tauv, mask=lanes == 0)
        return 0

    lax.fori_loop(0, ROWS_PER_W, row_loop, 0)
    pltpu.sync_copy(taubuf, tau_hbm.at[pl.ds(base, ROWS_PER_W)])


def _mask_body(z_ref, tau_ref, out_ref):
    i = pl.program_id(0)
    tau = tau_ref[pl.ds(i * BR, BR), :]
    z = z_ref[...]
    out_ref[...] = jnp.where(z >= tau, jnp.maximum(z, 0.0), 0.0)


def _dec_body(a_ref, b_ref, out_ref):
    k = pl.program_id(0)
    r = pl.program_id(1)
    rows = pl.ds(r * BM, BM)

    @pl.when(k == 0)
    def _():
        out_ref[rows, :] = jnp.zeros((BM, HIDDEN), jnp.float32)

    out_ref[rows, :] += jax.lax.dot_general(
        a_ref[...], b_ref[...],
        (((1,), (1,)), ((), ())),
        preferred_element_type=jnp.float32,
        precision=jax.lax.Precision.DEFAULT)


def kernel(x, W_enc, W_dec):
    z = pl.pallas_call(
        _enc_body,
        grid=(LATENT // LC, NTOK // BM),
        in_specs=[pl.BlockSpec((BM, HIDDEN), lambda j, r: (r, 0)),
                  pl.BlockSpec((LC, HIDDEN), lambda j, r: (j, 0))],
        out_specs=pl.BlockSpec((BM, LC), lambda j, r: (r, j)),
        out_shape=jax.ShapeDtypeStruct((NTOK, LATENT), jnp.float32),
    )(x, W_enc)

    mesh = plsc.VectorSubcoreMesh(core_axis_name="c", subcore_axis_name="s")
    tau = pl.kernel(
        _tau_body,
        out_type=jax.ShapeDtypeStruct((NTOK,), jnp.float32),
        mesh=mesh,
        compiler_params=pltpu.CompilerParams(needs_layout_passes=False),
        scratch_types=[
            pltpu.VMEM((LATENT,), jnp.float32),
            pltpu.VMEM((L * SEG,), jnp.float32),
            pltpu.VMEM((ROWS_PER_W,), jnp.float32),
        ],
    )(z)

    z_sparse = pl.pallas_call(
        _mask_body,
        grid=(NTOK // BR,),
        in_specs=[pl.BlockSpec((BR, LATENT), lambda i: (i, 0)),
                  pl.BlockSpec((NTOK, 1), lambda i: (0, 0))],
        out_specs=pl.BlockSpec((BR, LATENT), lambda i: (i, 0)),
        out_shape=jax.ShapeDtypeStruct((NTOK, LATENT), jnp.float32),
    )(z, tau.reshape(NTOK, 1))

    x_hat = pl.pallas_call(
        _dec_body,
        grid=(LATENT // KC, NTOK // BM),
        in_specs=[pl.BlockSpec((BM, KC), lambda k, r: (r, k)),
                  pl.BlockSpec((HIDDEN, KC), lambda k, r: (0, k))],
        out_specs=pl.BlockSpec((NTOK, HIDDEN), lambda k, r: (0, 0)),
        out_shape=jax.ShapeDtypeStruct((NTOK, HIDDEN), jnp.float32),
    )(z_sparse, W_dec)
    return x_hat, z_sparse


# speedup vs baseline: 2.0540x; 1.0833x over previous
import jax
import jax.numpy as jnp
from jax import lax
from jax.experimental import pallas as pl
from jax.experimental.pallas import tpu as pltpu
from jax.experimental.pallas import tpu_sc as plsc

HIDDEN = 2048
LATENT = 16384
TOPK = 64
NTOK = 2048

BM = 256
LC = 1024
BR = 128
KC = 512

NC = 2
NS = 16
L = 16
NW = NC * NS
ROWS_PER_W = NTOK // NW
NV = LATENT // L
MGROUP = 32
NM = NV // MGROUP
SEG = 32


def _enc_body(x_ref, w_ref, out_ref):
    out_ref[...] = jax.lax.dot_general(
        x_ref[...], w_ref[...],
        (((1,), (1,)), ((), ())),
        preferred_element_type=jnp.float32,
        precision=jax.lax.Precision.DEFAULT)


def _tau_body(z_hbm, tau_hbm, zrow, zrow2, cand, taubuf, sem, sem2):
    c = lax.axis_index("c")
    s = lax.axis_index("s")
    wid = s * NC + c
    base = wid * ROWS_PER_W
    lanes = lax.iota(jnp.int32, L)

    one = jnp.ones((L,), jnp.int32)
    zero = jnp.zeros((L,), jnp.int32)

    def lane_sum(v):
        for d in (1, 2, 4, 8):
            v = v + v[lanes ^ d]
        return v

    def process(zref, r):
        def build_m(g):
            def inner(k, acc):
                return jnp.maximum(acc, zref[pl.ds(k * L, L)])
            return plsc.parallel_loop(
                g * MGROUP, (g + 1) * MGROUP, unroll=8,
                carry=jnp.full((L,), -1e30, jnp.float32))(inner)
        m_regs = tuple(build_m(g) for g in range(NM))

        rv = m_regs[0]
        for g in range(1, NM):
            rv = jnp.maximum(rv, m_regs[g])
        for d in (1, 2, 4, 8):
            rv = jnp.maximum(rv, rv[lanes ^ d])
        rmaxv = rv

        def mb_body(i, st):
            lo, hi = st
            mid = 0.5 * (lo + hi)
            acc = zero
            for g in range(NM):
                acc = acc + jnp.where(m_regs[g] >= mid, one, zero)
            pred = lane_sum(acc) >= TOPK
            return (jnp.where(pred, mid, lo), jnp.where(pred, hi, mid))

        tau0v, _ = lax.fori_loop(
            0, 12, mb_body, (jnp.zeros((L,), jnp.float32), rmaxv + 0.5))

        for v in range(SEG):
            cand[pl.ds(v * L, L)] = jnp.zeros((L,), jnp.float32)

        segbase = lanes * SEG

        def compact(v, cnt):
            x = zref[pl.ds(v * L, L)]
            mask = x >= tau0v
            ok = jnp.logical_and(mask, cnt < SEG)
            plsc.store_scatter(cand, [segbase + cnt], x, mask=ok)
            return cnt + jnp.where(mask, one, zero)

        plsc.parallel_loop(0, NV, unroll=8, carry=zero)(compact)

        c_regs = tuple(cand[pl.ds(v * L, L)] for v in range(SEG))

        def d_body(i, st):
            lo, hi = st
            mid = 0.5 * (lo + hi)
            acc = zero
            for v in range(SEG):
                acc = acc + jnp.where(c_regs[v] >= mid, one, zero)
            pred = lane_sum(acc) >= TOPK
            return (jnp.where(pred, mid, lo), jnp.where(pred, hi, mid))

        tauv, _ = lax.fori_loop(0, 30, d_body, (tau0v, rmaxv + 0.5))

        plsc.store_scatter(taubuf, [jnp.full((L,), r, jnp.int32)],
                           tauv, mask=lanes == 0)

    def nxt(r):
        return jnp.minimum(base + r, NTOK - 1)

    pltpu.async_copy(z_hbm.at[base], zrow, sem)

    def pair(p, carry):
        r0 = 2 * p
        pltpu.async_copy(z_hbm.at[nxt(r0 + 1)], zrow2, sem2)
        pltpu.make_async_copy(z_hbm.at[base], zrow, sem).wait()
        process(zrow, r0)
        pltpu.async_copy(z_hbm.at[nxt(r0 + 2)], zrow, sem)
        pltpu.make_async_copy(z_hbm.at[base], zrow2, sem2).wait()
        process(zrow2, r0 + 1)
        return 0

    lax.fori_loop(0, ROWS_PER_W // 2, pair, 0)
    pltpu.make_async_copy(z_hbm.at[base], zrow, sem).wait()
    pltpu.sync_copy(taubuf, tau_hbm.at[pl.ds(base, ROWS_PER_W)])


def _mask_body(z_ref, tau_ref, out_ref):
    i = pl.program_id(0)
    tau = tau_ref[pl.ds(i * BR, BR), :]
    z = z_ref[...]
    out_ref[...] = jnp.where(z >= tau, jnp.maximum(z, 0.0), 0.0)


def _dec_body(a_ref, b_ref, out_ref):
    k = pl.program_id(0)
    r = pl.program_id(1)
    rows = pl.ds(r * BM, BM)

    @pl.when(k == 0)
    def _():
        out_ref[rows, :] = jnp.zeros((BM, HIDDEN), jnp.float32)

    out_ref[rows, :] += jax.lax.dot_general(
        a_ref[...], b_ref[...],
        (((1,), (1,)), ((), ())),
        preferred_element_type=jnp.float32,
        precision=jax.lax.Precision.DEFAULT)


def kernel(x, W_enc, W_dec):
    z = pl.pallas_call(
        _enc_body,
        grid=(LATENT // LC, NTOK // BM),
        in_specs=[pl.BlockSpec((BM, HIDDEN), lambda j, r: (r, 0)),
                  pl.BlockSpec((LC, HIDDEN), lambda j, r: (j, 0))],
        out_specs=pl.BlockSpec((BM, LC), lambda j, r: (r, j)),
        out_shape=jax.ShapeDtypeStruct((NTOK, LATENT), jnp.float32),
    )(x, W_enc)

    mesh = plsc.VectorSubcoreMesh(core_axis_name="c", subcore_axis_name="s")
    tau = pl.kernel(
        _tau_body,
        out_type=jax.ShapeDtypeStruct((NTOK,), jnp.float32),
        mesh=mesh,
        compiler_params=pltpu.CompilerParams(needs_layout_passes=False),
        scratch_types=[
            pltpu.VMEM((LATENT,), jnp.float32),
            pltpu.VMEM((LATENT,), jnp.float32),
            pltpu.VMEM((L * SEG,), jnp.float32),
            pltpu.VMEM((ROWS_PER_W,), jnp.float32),
            pltpu.SemaphoreType.DMA,
            pltpu.SemaphoreType.DMA,
        ],
    )(z)

    z_sparse = pl.pallas_call(
        _mask_body,
        grid=(NTOK // BR,),
        in_specs=[pl.BlockSpec((BR, LATENT), lambda i: (i, 0)),
                  pl.BlockSpec((NTOK, 1), lambda i: (0, 0))],
        out_specs=pl.BlockSpec((BR, LATENT), lambda i: (i, 0)),
        out_shape=jax.ShapeDtypeStruct((NTOK, LATENT), jnp.float32),
    )(z, tau.reshape(NTOK, 1))

    x_hat = pl.pallas_call(
        _dec_body,
        grid=(LATENT // KC, NTOK // BM),
        in_specs=[pl.BlockSpec((BM, KC), lambda k, r: (r, k)),
                  pl.BlockSpec((HIDDEN, KC), lambda k, r: (0, k))],
        out_specs=pl.BlockSpec((NTOK, HIDDEN), lambda k, r: (0, 0)),
        out_shape=jax.ShapeDtypeStruct((NTOK, HIDDEN), jnp.float32),
    )(z_sparse, W_dec)
    return x_hat, z_sparse
